# baseline (speedup 1.0000x reference)
import jax
import jax.numpy as jnp
from jax.experimental import pallas as pl
from jax.experimental.pallas import tpu as pltpu

EMB = 256
N_GRAPHS = 256
N_LAYERS = 5
N_ATOM_TYPES = 118
OUT_PAD = 128


def _mlp_softmax_kernel(gf_ref, w_ref, b_ref, out_ref):
    logits = jnp.dot(gf_ref[...], w_ref[...], preferred_element_type=jnp.float32)
    logits = logits + b_ref[...]
    m = jnp.max(logits, axis=-1, keepdims=True)
    e = jnp.exp(logits - m)
    s = jnp.sum(e, axis=-1, keepdims=True)
    out_ref[...] = e / s


def kernel(atom_feat, bond_feat, edge_index, graph_ids, atom_table, edge_table,
           gnn_W, gnn_b, mlp_W, mlp_b):
    x = jnp.take(atom_table, atom_feat, axis=0)
    e = jnp.take(edge_table, bond_feat, axis=0)
    src = edge_index[0]
    dst = edge_index[1]
    for i in range(N_LAYERS):
        msg = jax.nn.relu(x[src] + e)
        agg = jax.ops.segment_sum(msg, dst, num_segments=x.shape[0])
        x = jax.nn.relu((x + agg) @ gnn_W[i] + gnn_b[i])
    graph_feat = jax.ops.segment_sum(x, graph_ids, num_segments=N_GRAPHS)

    w_pad = jnp.zeros((EMB, OUT_PAD), jnp.float32).at[:, :N_ATOM_TYPES].set(mlp_W)
    b_pad = jnp.full((1, OUT_PAD), -1e30, jnp.float32).at[0, :N_ATOM_TYPES].set(mlp_b)
    probs = pl.pallas_call(
        _mlp_softmax_kernel,
        out_shape=jax.ShapeDtypeStruct((N_GRAPHS, OUT_PAD), jnp.float32),
    )(graph_feat, w_pad, b_pad)
    return probs[:, :N_ATOM_TYPES]


# speedup vs baseline: 3.8269x; 3.8269x over previous
import functools

import jax
import jax.numpy as jnp
from jax import lax
from jax.experimental import pallas as pl
from jax.experimental.pallas import tpu as pltpu
from jax.experimental.pallas import tpu_sc as plsc

EMB = 256
HALF = 128
NODES = 10000
NP = 10240
EDGES = 160000
EP = 163840
CHUNK = 128
NCHUNK = 80
PHASE = 40
N_GRAPHS = 256
N_LAYERS = 5
N_ATOM_TYPES = 118
OUT_PAD = 128
ROWS_PER_TILE = NP // 16
EDGES_PER_TILE = EP // 16

def _sc_embed_body(tab, af3, srcp, bondp, xh, idx_out,
                   af_buf, rows, srcb, bondb, idxb, sem):
    c = lax.axis_index("c")
    s = lax.axis_index("s")
    pltpu.sync_copy(af3.at[s], af_buf)
    for j in range(5):
        pltpu.async_copy(tab.at[c].at[af_buf.at[j]], rows, sem).wait()
        pltpu.sync_copy(rows, xh.at[c, pl.ds(s * ROWS_PER_TILE + j * CHUNK, CHUNK)])

    @pl.when(c == 0)
    def _():
        base = s * EDGES_PER_TILE
        pltpu.sync_copy(srcp.at[pl.ds(base, EDGES_PER_TILE)], srcb)
        pltpu.sync_copy(bondp.at[pl.ds(base, EDGES_PER_TILE)], bondb)

        def it(i, carry):
            sv = srcb[pl.ds(i * 16, 16)]
            bv = bondb[pl.ds(i * 16, 16)]
            idxb[pl.ds(i * 16, 16)] = bv * NP + sv
            return carry

        lax.fori_loop(0, EDGES_PER_TILE // 16, it, 0)
        pltpu.sync_copy(idxb, idx_out.at[pl.ds(base, EDGES_PER_TILE)])


@functools.lru_cache(maxsize=None)
def _make_sc_embed():
    mesh = plsc.VectorSubcoreMesh(core_axis_name="c", subcore_axis_name="s")
    return pl.kernel(
        _sc_embed_body,
        out_type=(jax.ShapeDtypeStruct((2, NP, HALF), jnp.float32),
                  jax.ShapeDtypeStruct((EP,), jnp.int32)),
        mesh=mesh,
        scratch_types=[
            pltpu.VMEM((5, CHUNK), jnp.int32),
            pltpu.VMEM((CHUNK, HALF), jnp.float32),
            pltpu.VMEM((EDGES_PER_TILE,), jnp.int32),
            pltpu.VMEM((EDGES_PER_TILE,), jnp.int32),
            pltpu.VMEM((EDGES_PER_TILE,), jnp.int32),
            pltpu.SemaphoreType.DMA,
        ],
    )


def _sc_msg_body(Y, h, idx3, dst3, agg,
                 spmem, idxb, dstb, ra, rb, sema, semb):
    c = lax.axis_index("c")
    s = lax.axis_index("s")
    base = s * ROWS_PER_TILE
    pltpu.sync_copy(h.at[c, pl.ds(base, ROWS_PER_TILE)],
                    spmem.at[pl.ds(base, ROWS_PER_TILE)])
    plsc.subcore_barrier()

    Yc = Y.at[c]

    def fire(g, buf, sem):
        pltpu.async_copy(Yc.at[idxb.at[g]], buf, sem)

    def drain(buf, sem):
        pltpu.make_async_copy(Yc.at[pl.ds(0, CHUNK)], buf, sem).wait()

    def scat(g, buf):
        pltpu.sync_copy(buf, spmem.at[dstb.at[g]], add=True)

    for p in range(2):
        pltpu.sync_copy(idx3.at[s, pl.ds(p * PHASE, PHASE)], idxb)
        pltpu.sync_copy(dst3.at[s, pl.ds(p * PHASE, PHASE)], dstb)
        fire(0, ra, sema)

        def gbody(g, carry):
            @pl.when(g % 2 == 0)
            def _():
                @pl.when(g + 1 < PHASE)
                def _():
                    fire(g + 1, rb, semb)
                drain(ra, sema)
                scat(g, ra)

            @pl.when(g % 2 == 1)
            def _():
                @pl.when(g + 1 < PHASE)
                def _():
                    fire(g + 1, ra, sema)
                drain(rb, semb)
                scat(g, rb)

            return carry

        lax.fori_loop(0, PHASE, gbody, 0)

    plsc.subcore_barrier()
    pltpu.sync_copy(spmem.at[pl.ds(base, ROWS_PER_TILE)],
                    agg.at[c, pl.ds(base, ROWS_PER_TILE)])


@functools.lru_cache(maxsize=None)
def _make_sc_msg():
    mesh = plsc.VectorSubcoreMesh(core_axis_name="c", subcore_axis_name="s")
    return pl.kernel(
        _sc_msg_body,
        out_type=jax.ShapeDtypeStruct((2, NP, HALF), jnp.float32),
        mesh=mesh,
        scratch_types=[
            pltpu.VMEM_SHARED((NP, HALF), jnp.float32),
            pltpu.VMEM((PHASE, CHUNK), jnp.int32),
            pltpu.VMEM((PHASE, CHUNK), jnp.int32),
            pltpu.VMEM((CHUNK, HALF), jnp.float32),
            pltpu.VMEM((CHUNK, HALF), jnp.float32),
            pltpu.SemaphoreType.DMA,
            pltpu.SemaphoreType.DMA,
        ],
    )


def _prep_body(x_ref, e_ref, y_ref):
    xfull = jnp.concatenate([x_ref[0], x_ref[1]], axis=-1)
    y = jnp.maximum(xfull[None] + e_ref[...][:, None, :], 0.0)
    y_ref[0] = y[:, :, :HALF]
    y_ref[1] = y[:, :, HALF:]


def _update_body(agg_ref, w_ref, b_ref, e_ref, h_ref, y_ref):
    s = jnp.concatenate([agg_ref[0], agg_ref[1]], axis=-1)
    h = jnp.dot(s, w_ref[...], preferred_element_type=jnp.float32)
    h = jnp.maximum(h + b_ref[...], 0.0)
    h_ref[0] = h[:, :HALF]
    h_ref[1] = h[:, HALF:]
    y = jnp.maximum(h[None] + e_ref[...][:, None, :], 0.0)
    y_ref[0] = y[:, :, :HALF]
    y_ref[1] = y[:, :, HALF:]


def _final_body(agg_ref, w_ref, b_ref, gid_ref, mw_ref, mb_ref, out_ref, seg_ref):
    i = pl.program_id(0)

    @pl.when(i == 0)
    def _():
        seg_ref[...] = jnp.zeros_like(seg_ref)

    s = jnp.concatenate([agg_ref[0], agg_ref[1]], axis=-1)
    h = jnp.dot(s, w_ref[...], preferred_element_type=jnp.float32)
    h = jnp.maximum(h + b_ref[...], 0.0)
    gid = gid_ref[...]
    iota_l = lax.broadcasted_iota(jnp.int32, gid.shape, 1)
    oh = jnp.concatenate([(gid == iota_l).astype(jnp.float32),
                          (gid == iota_l + 128).astype(jnp.float32)],
                         axis=-1)
    seg_ref[...] += lax.dot_general(oh, h, (((0,), (0,)), ((), ())),
                                    preferred_element_type=jnp.float32)

    @pl.when(i == pl.num_programs(0) - 1)
    def _():
        logits = jnp.dot(seg_ref[...], mw_ref[...],
                         preferred_element_type=jnp.float32) + mb_ref[...]
        m = jnp.max(logits, axis=-1, keepdims=True)
        ex = jnp.exp(logits - m)
        out_ref[...] = ex / jnp.sum(ex, axis=-1, keepdims=True)


_RU = 1024
_RF = 1000


def _tc_prep(xh, e_table):
    return pl.pallas_call(
        _prep_body,
        grid=(NP // _RU,),
        in_specs=[pl.BlockSpec((2, _RU, HALF), lambda i: (0, i, 0)),
                  pl.BlockSpec((4, EMB), lambda i: (0, 0))],
        out_specs=pl.BlockSpec((2, 4, _RU, HALF), lambda i: (0, 0, i, 0)),
        out_shape=jax.ShapeDtypeStruct((2, 4, NP, HALF), jnp.float32),
    )(xh, e_table)


def _tc_update(agg, w, b, e_table):
    return pl.pallas_call(
        _update_body,
        grid=(NP // _RU,),
        in_specs=[pl.BlockSpec((2, _RU, HALF), lambda i: (0, i, 0)),
                  pl.BlockSpec((EMB, EMB), lambda i: (0, 0)),
                  pl.BlockSpec((1, EMB), lambda i: (0, 0)),
                  pl.BlockSpec((4, EMB), lambda i: (0, 0))],
        out_specs=[pl.BlockSpec((2, _RU, HALF), lambda i: (0, i, 0)),
                   pl.BlockSpec((2, 4, _RU, HALF), lambda i: (0, 0, i, 0))],
        out_shape=[jax.ShapeDtypeStruct((2, NP, HALF), jnp.float32),
                   jax.ShapeDtypeStruct((2, 4, NP, HALF), jnp.float32)],
    )(agg, w, b, e_table)


def _tc_final(agg, w, b, gidb, mw, mb):
    return pl.pallas_call(
        _final_body,
        grid=(NODES // _RF,),
        in_specs=[pl.BlockSpec((2, _RF, HALF), lambda i: (0, i, 0)),
                  pl.BlockSpec((EMB, EMB), lambda i: (0, 0)),
                  pl.BlockSpec((1, EMB), lambda i: (0, 0)),
                  pl.BlockSpec((_RF, HALF), lambda i: (i, 0)),
                  pl.BlockSpec((EMB, OUT_PAD), lambda i: (0, 0)),
                  pl.BlockSpec((1, OUT_PAD), lambda i: (0, 0))],
        out_specs=pl.BlockSpec((N_GRAPHS, OUT_PAD), lambda i: (0, 0)),
        out_shape=jax.ShapeDtypeStruct((N_GRAPHS, OUT_PAD), jnp.float32),
        scratch_shapes=[pltpu.VMEM((N_GRAPHS, EMB), jnp.float32)],
    )(agg, w, b, gidb, mw, mb)


def kernel(atom_feat, bond_feat, edge_index, graph_ids, atom_table, edge_table,
           gnn_W, gnn_b, mlp_W, mlp_b):
    src = edge_index[0].astype(jnp.int32)
    dst = edge_index[1].astype(jnp.int32)
    bond = bond_feat.astype(jnp.int32)
    epad = EP - EDGES
    srcp = jnp.concatenate([src, jnp.zeros((epad,), jnp.int32)])
    bondp = jnp.concatenate([bond, jnp.zeros((epad,), jnp.int32)])
    dstp = jnp.concatenate([dst, jnp.full((epad,), NP - 1, jnp.int32)])
    dst3 = dstp.reshape(16, NCHUNK, CHUNK)
    afp = jnp.concatenate([atom_feat.astype(jnp.int32),
                           jnp.zeros((NP - NODES,), jnp.int32)])
    af3 = afp.reshape(16, 5, CHUNK)
    tab2 = jnp.stack([atom_table[:, :HALF], atom_table[:, HALF:]])
    gidb = jnp.broadcast_to(graph_ids.astype(jnp.int32)[:, None],
                            (NODES, HALF))
    mw_pad = jnp.zeros((EMB, OUT_PAD), jnp.float32).at[:, :N_ATOM_TYPES].set(mlp_W)
    mb_pad = jnp.full((1, OUT_PAD), -1e30, jnp.float32).at[0, :N_ATOM_TYPES].set(mlp_b)

    xh, idx_flat = _make_sc_embed()(tab2, af3, srcp, bondp)
    idx3 = idx_flat.reshape(16, NCHUNK, CHUNK)

    y4 = _tc_prep(xh, edge_table)
    h = xh
    for l in range(N_LAYERS - 1):
        y = y4.reshape(2, 4 * NP, HALF)
        agg = _make_sc_msg()(y, h, idx3, dst3)
        h, y4 = _tc_update(agg, gnn_W[l], gnn_b[l].reshape(1, EMB), edge_table)
    y = y4.reshape(2, 4 * NP, HALF)
    agg = _make_sc_msg()(y, h, idx3, dst3)
    probs = _tc_final(agg, gnn_W[4], gnn_b[4].reshape(1, EMB), gidb,
                      mw_pad, mb_pad)
    return probs[:, :N_ATOM_TYPES]
